# Initial kernel scaffold; baseline (speedup 1.0000x reference)
import jax
import jax.numpy as jnp
from jax.experimental import pallas as pl


def kernel(img_feat, word_emb, wc, wl, bl, wconv):
    raise NotImplementedError("write your pallas kernel here")


# speedup vs baseline: 1.8894x; 1.8894x over previous
import functools

import jax
import jax.numpy as jnp
from jax.experimental import pallas as pl
from jax.experimental.pallas import tpu as pltpu


def _softmax(x, axis):
    m = jnp.max(x, axis=axis, keepdims=True)
    e = jnp.exp(x - m)
    return e * pl.reciprocal(jnp.sum(e, axis=axis, keepdims=True), approx=True)


def _co_attn_kernel(img_ref, words_ref, wc_ref, wl_ref, bl_ref, wconv_ref,
                    out_ref, *cols, H, W, ITEMS):
    IT = ITEMS
    C = img_ref.shape[1]
    N = img_ref.shape[2]
    T = words_ref.shape[2]
    cin2 = 2 * C
    T2 = 2 * T

    img_all = img_ref[...].reshape(IT * C, N).astype(jnp.bfloat16)
    words_all = jnp.concatenate(
        [words_ref[i] for i in range(IT)], axis=1).astype(jnp.bfloat16)

    src_all = jnp.dot(wc_ref[...], words_all,
                      preferred_element_type=jnp.float32)
    src_b = src_all.astype(jnp.bfloat16)

    wtil_all = jax.lax.dot_general(words_all, wl_ref[...],
                                   (((0,), (1,)), ((), ())),
                                   preferred_element_type=jnp.float32)
    wtil_all = wtil_all + bl_ref[...]

    wtil_bs = []
    m_list = []
    for i in range(IT):
        wtil_bi = wtil_all[i * T:(i + 1) * T].astype(jnp.bfloat16)
        wtil_bs.append(wtil_bi)
        m_list.append(jax.lax.dot_general(wtil_bi, img_all[i * C:(i + 1) * C],
                                          (((1,), (1,)), ((), ())),
                                          preferred_element_type=jnp.float32))

    zc = jnp.zeros((C, T), jnp.bfloat16)
    lhs_attn = jnp.concatenate(
        [jnp.concatenate([src_b[:, i * T:(i + 1) * T] if j == i else zc
                          for j in range(IT)], axis=1)
         for i in range(IT)], axis=0)
    attn_all = jax.lax.dot_general(lhs_attn, img_all, (((0,), (0,)), ((), ())),
                                   preferred_element_type=jnp.float32)

    l_cols = []
    rhs_items = []
    for i in range(IT):
        attn_i = _softmax(attn_all[i * T:(i + 1) * T], axis=0)
        alpha_i = jnp.transpose(
            _softmax(m_list[i], axis=0)).astype(jnp.bfloat16)
        s_i = src_b[:, i * T:(i + 1) * T]
        l_cols.append(jnp.concatenate(
            [jnp.concatenate([alpha_i, zc], axis=1),
             jnp.concatenate([zc, s_i], axis=1)], axis=0))
        rhs_items.append(jnp.concatenate(
            [wtil_bs[i], attn_i.astype(jnp.bfloat16)], axis=0))
    l_all = jnp.concatenate(l_cols, axis=1)

    g_taps = []
    for tap in range(9):
        gt = jnp.dot(wconv_ref[pl.ds(tap * cin2, cin2), :], l_all,
                     preferred_element_type=jnp.float32)
        g_taps.append(gt.astype(jnp.bfloat16))

    n_idx = jax.lax.broadcasted_iota(jnp.int32, (1, N), 1)
    h_idx = n_idx // W
    w_idx = n_idx % W
    SHIFT_TAPS = [(dy, dx) for dy in (-1, 0, 1) for dx in (-1, 0, 1)
                  if (dy, dx) != (0, 0)]
    zeros_edge = jnp.zeros((T2, W), jnp.bfloat16)
    for i in range(IT):
        r = rhs_items[i]
        col_ref = cols[i]
        for t8, (dy, dx) in enumerate(SHIFT_TAPS):
            k = dy * W + dx
            shifted = pltpu.roll(r, shift=(-k) % N, axis=1)
            rows = pl.ds(t8 * T2, T2)
            if dx == 0:
                col_ref[rows, :] = shifted
                if dy < 0:
                    col_ref[rows, pl.ds(0, W)] = zeros_edge
                else:
                    col_ref[rows, pl.ds(N - W, W)] = zeros_edge
            else:
                valid = ((h_idx + dy >= 0) & (h_idx + dy < H) &
                         (w_idx + dx >= 0) & (w_idx + dx < W))
                col_ref[rows, :] = jnp.where(valid, shifted, 0)

    for i in range(IT):
        g_i = jnp.concatenate(
            [g_taps[(dy + 1) * 3 + (dx + 1)][:, i * T2:(i + 1) * T2]
             for (dy, dx) in SHIFT_TAPS], axis=1)
        conv = jnp.dot(g_i, cols[i][...],
                       preferred_element_type=jnp.float32)
        conv = conv + jnp.dot(g_taps[4][:, i * T2:(i + 1) * T2], rhs_items[i],
                              preferred_element_type=jnp.float32)
        out = conv[:C] * jax.nn.sigmoid(conv[C:])
        out_ref[i] = out.astype(out_ref.dtype)


def kernel(img_feat, word_emb, wc, wl, bl, wconv):
    B, C, H, W = img_feat.shape
    _, D, T = word_emb.shape
    N = H * W
    cin2 = 2 * C

    img_flat = img_feat.reshape(B, C, N)
    wc_b = wc.astype(jnp.bfloat16)
    wl_b = wl.astype(jnp.bfloat16)
    bl_1n = bl.reshape(1, N).astype(jnp.float32)
    wconv_b = jnp.transpose(wconv, (2, 3, 0, 1)).reshape(9 * cin2, cin2)
    wconv_b = wconv_b.astype(jnp.bfloat16)

    ITEMS = 4
    while B % ITEMS:
        ITEMS //= 2
    out_flat = pl.pallas_call(
        functools.partial(_co_attn_kernel, H=H, W=W, ITEMS=ITEMS),
        out_shape=jax.ShapeDtypeStruct((B, C, N), jnp.float32),
        grid=(B // ITEMS,),
        in_specs=[
            pl.BlockSpec((ITEMS, C, N), lambda b: (b, 0, 0)),
            pl.BlockSpec((ITEMS, D, T), lambda b: (b, 0, 0)),
            pl.BlockSpec((C, D), lambda b: (0, 0)),
            pl.BlockSpec((N, D), lambda b: (0, 0)),
            pl.BlockSpec((1, N), lambda b: (0, 0)),
            pl.BlockSpec((9 * cin2, cin2), lambda b: (0, 0)),
        ],
        out_specs=pl.BlockSpec((ITEMS, C, N), lambda b: (b, 0, 0)),
        scratch_shapes=[pltpu.VMEM((8 * 2 * T, N), jnp.bfloat16)
                        for _ in range(ITEMS)],
        compiler_params=pltpu.CompilerParams(
            dimension_semantics=("parallel",),
            vmem_limit_bytes=64 * 1024 * 1024,
        ),
    )(img_flat, word_emb, wc_b, wl_b, bl_1n, wconv_b)
    return out_flat.reshape(B, C, H, W)


# speedup vs baseline: 1.9618x; 1.0383x over previous
import functools

import jax
import jax.numpy as jnp
from jax.experimental import pallas as pl
from jax.experimental.pallas import tpu as pltpu


def _softmax(x, axis):
    m = jnp.max(x, axis=axis, keepdims=True)
    e = jnp.exp(x - m)
    return e * pl.reciprocal(jnp.sum(e, axis=axis, keepdims=True), approx=True)


def _co_attn_kernel(img_ref, words_ref, wc_ref, wl_ref, bl_ref, wconv_ref,
                    out_ref, *cols, H, W, ITEMS):
    IT = ITEMS
    C = img_ref.shape[1]
    N = img_ref.shape[2]
    T = words_ref.shape[2]
    cin2 = 2 * C
    T2 = 2 * T

    img_all = img_ref[...].reshape(IT * C, N).astype(jnp.bfloat16)
    words_all = jnp.concatenate(
        [words_ref[i] for i in range(IT)], axis=1).astype(jnp.bfloat16)

    src_all = jnp.dot(wc_ref[...], words_all,
                      preferred_element_type=jnp.float32)
    src_b = src_all.astype(jnp.bfloat16)

    wtil_all = jax.lax.dot_general(words_all, wl_ref[...],
                                   (((0,), (1,)), ((), ())),
                                   preferred_element_type=jnp.float32)
    wtil_all = wtil_all + bl_ref[...]

    wtil_bs = []
    m_list = []
    for i in range(IT):
        wtil_bi = wtil_all[i * T:(i + 1) * T].astype(jnp.bfloat16)
        wtil_bs.append(wtil_bi)
        m_list.append(jax.lax.dot_general(wtil_bi, img_all[i * C:(i + 1) * C],
                                          (((1,), (1,)), ((), ())),
                                          preferred_element_type=jnp.float32))

    zc = jnp.zeros((C, T), jnp.bfloat16)
    lhs_attn = jnp.concatenate(
        [jnp.concatenate([src_b[:, i * T:(i + 1) * T] if j == i else zc
                          for j in range(IT)], axis=1)
         for i in range(IT)], axis=0)
    attn_all = jax.lax.dot_general(lhs_attn, img_all, (((0,), (0,)), ((), ())),
                                   preferred_element_type=jnp.float32)

    l_cols = []
    rhs_items = []
    for i in range(IT):
        attn_i = _softmax(attn_all[i * T:(i + 1) * T], axis=0)
        alpha_i = jnp.transpose(
            _softmax(m_list[i], axis=0)).astype(jnp.bfloat16)
        s_i = src_b[:, i * T:(i + 1) * T]
        l_cols.append(jnp.concatenate(
            [jnp.concatenate([alpha_i, zc], axis=1),
             jnp.concatenate([zc, s_i], axis=1)], axis=0))
        rhs_items.append(jnp.concatenate(
            [wtil_bs[i], attn_i.astype(jnp.bfloat16)], axis=0))
    l_all = jnp.concatenate(l_cols, axis=1)

    g_taps = []
    for tap in range(9):
        gt = jnp.dot(wconv_ref[pl.ds(tap * cin2, cin2), :], l_all,
                     preferred_element_type=jnp.float32)
        g_taps.append(gt.astype(jnp.bfloat16))

    n_idx = jax.lax.broadcasted_iota(jnp.int32, (1, N), 1)
    h_idx = n_idx // W
    w_idx = n_idx % W
    SHIFT_TAPS = [(dy, dx) for dy in (-1, 0, 1) for dx in (-1, 0, 1)
                  if (dy, dx) != (0, 0)]
    zeros_edge = jnp.zeros((T2, W), jnp.bfloat16)
    for i in range(IT):
        r = rhs_items[i]
        col_ref = cols[i]
        for t8, (dy, dx) in enumerate(SHIFT_TAPS):
            k = dy * W + dx
            shifted = pltpu.roll(r, shift=(-k) % N, axis=1)
            rows = pl.ds(t8 * T2, T2)
            if dx == 0:
                col_ref[rows, :] = shifted
                if dy < 0:
                    col_ref[rows, pl.ds(0, W)] = zeros_edge
                else:
                    col_ref[rows, pl.ds(N - W, W)] = zeros_edge
            else:
                valid = ((h_idx + dy >= 0) & (h_idx + dy < H) &
                         (w_idx + dx >= 0) & (w_idx + dx < W))
                col_ref[rows, :] = jnp.where(valid, shifted, 0)

    for i in range(IT):
        g_i = jnp.concatenate(
            [g_taps[(dy + 1) * 3 + (dx + 1)][:, i * T2:(i + 1) * T2]
             for (dy, dx) in SHIFT_TAPS], axis=1)
        conv = jnp.dot(g_i, cols[i][...],
                       preferred_element_type=jnp.float32)
        conv = conv + jnp.dot(g_taps[4][:, i * T2:(i + 1) * T2], rhs_items[i],
                              preferred_element_type=jnp.float32)
        out = conv[:C] * jax.nn.sigmoid(conv[C:])
        out_ref[i] = out.astype(out_ref.dtype)


def kernel(img_feat, word_emb, wc, wl, bl, wconv):
    B, C, H, W = img_feat.shape
    _, D, T = word_emb.shape
    N = H * W
    cin2 = 2 * C

    img_flat = img_feat.reshape(B, C, N)
    wc_b = wc.astype(jnp.bfloat16)
    wl_b = wl.astype(jnp.bfloat16)
    bl_1n = bl.reshape(1, N).astype(jnp.float32)
    wconv_b = jnp.transpose(wconv, (2, 3, 0, 1)).reshape(9 * cin2, cin2)
    wconv_b = wconv_b.astype(jnp.bfloat16)

    ITEMS = 8
    while B % ITEMS:
        ITEMS //= 2
    out_flat = pl.pallas_call(
        functools.partial(_co_attn_kernel, H=H, W=W, ITEMS=ITEMS),
        out_shape=jax.ShapeDtypeStruct((B, C, N), jnp.float32),
        grid=(B // ITEMS,),
        in_specs=[
            pl.BlockSpec((ITEMS, C, N), lambda b: (b, 0, 0)),
            pl.BlockSpec((ITEMS, D, T), lambda b: (b, 0, 0)),
            pl.BlockSpec((C, D), lambda b: (0, 0)),
            pl.BlockSpec((N, D), lambda b: (0, 0)),
            pl.BlockSpec((1, N), lambda b: (0, 0)),
            pl.BlockSpec((9 * cin2, cin2), lambda b: (0, 0)),
        ],
        out_specs=pl.BlockSpec((ITEMS, C, N), lambda b: (b, 0, 0)),
        scratch_shapes=[pltpu.VMEM((8 * 2 * T, N), jnp.bfloat16)
                        for _ in range(ITEMS)],
        compiler_params=pltpu.CompilerParams(
            dimension_semantics=("parallel",),
            vmem_limit_bytes=64 * 1024 * 1024,
        ),
    )(img_flat, word_emb, wc_b, wl_b, bl_1n, wconv_b)
    return out_flat.reshape(B, C, H, W)
